# baseline (speedup 1.0000x reference)
import jax
import jax.numpy as jnp
from jax import lax
from jax.experimental import pallas as pl
from jax.experimental.pallas import tpu as pltpu
from jax.experimental.pallas import tpu_sc as plsc

VOCAB = 1000000
EMBED_DIM = 64
BATCH = 16384
HIST = 50

_NC = 2
_NS = 16
_NW = _NC * _NS

_N_ROWS = BATCH * HIST
_PER_W = _N_ROWS // _NW
_GRP = 256
_NGRP = _PER_W // _GRP
_NBUF = 6


def _sc_gather(idx, table):
    mesh = plsc.VectorSubcoreMesh(core_axis_name="c", subcore_axis_name="s")

    scratch = [pltpu.VMEM((_NGRP, _GRP), jnp.int32)]
    scratch += [pltpu.VMEM((_GRP, EMBED_DIM), jnp.float32)] * _NBUF
    scratch += [pltpu.SemaphoreType.DMA] * (2 * _NBUF)

    @pl.kernel(
        out_type=jax.ShapeDtypeStruct((_N_ROWS, EMBED_DIM), jnp.float32),
        mesh=mesh,
        compiler_params=pltpu.CompilerParams(use_tc_tiling_on_sc=False),
        scratch_types=scratch,
    )
    def k(idx_hbm, table_hbm, out_hbm, idx_v, *rest):
        bufs = rest[:_NBUF]
        gsem = rest[_NBUF:2 * _NBUF]
        wsem = rest[2 * _NBUF:]
        wid = lax.axis_index("s") * _NC + lax.axis_index("c")
        base = wid * _PER_W
        pltpu.sync_copy(idx_hbm.at[wid], idx_v)

        def wait_gather(b):
            pltpu.make_async_copy(
                table_hbm.at[pl.ds(0, _GRP)], bufs[b], gsem[b]).wait()

        def wait_write(b):
            pltpu.make_async_copy(
                bufs[b], out_hbm.at[pl.ds(0, _GRP)], wsem[b]).wait()

        for b in range(_NBUF):
            pltpu.async_copy(table_hbm.at[idx_v.at[b]], bufs[b], gsem[b])

        def outer(g, carry):
            for b in range(_NBUF):
                j = g * _NBUF + b
                wait_gather(b)
                pltpu.async_copy(
                    bufs[b], out_hbm.at[pl.ds(base + j * _GRP, _GRP)],
                    wsem[b])
                bp = (b - 1) % _NBUF
                jp = j - 1
                jn = jp + _NBUF

                @pl.when(jp >= 0)
                def _():
                    wait_write(bp)

                @pl.when(jnp.logical_and(jp >= 0, jn < _NGRP))
                def _():
                    pltpu.async_copy(
                        table_hbm.at[idx_v.at[jn]], bufs[bp], gsem[bp])
            return carry

        lax.fori_loop(0, _NGRP // _NBUF, outer, 0)
        for j in range(_NGRP - _NGRP % _NBUF, _NGRP):
            b = j % _NBUF
            wait_gather(b)
            pltpu.async_copy(
                bufs[b], out_hbm.at[pl.ds(base + j * _GRP, _GRP)], wsem[b])
            bp = (b - 1) % _NBUF
            wait_write(bp)
            jn = j - 1 + _NBUF
            if jn < _NGRP:
                pltpu.async_copy(
                    table_hbm.at[idx_v.at[jn]], bufs[bp], gsem[bp])
        wait_write((_NGRP - 1) % _NBUF)

    return k(idx, table)


def kernel(x, table):
    idx = x.reshape(_NW, _NGRP, _GRP).astype(jnp.int32)
    out = _sc_gather(idx, table)
    return out.reshape(BATCH, HIST, EMBED_DIM)
